# baseline (speedup 1.0000x reference)
import functools

import jax
import jax.numpy as jnp
from jax.experimental import pallas as pl
from jax.experimental.pallas import tpu as pltpu

_IN = 768
_OUT = 768
_NP = 4
_NE = 8
_BT = 1024
_SUB = 4


def _mask_top1(h):
    f32 = jnp.float32
    bf16 = jnp.bfloat16
    ne = _NP * _NE
    eqs = []
    for g in range(_NP):
        hg = h[:, g * _NE:(g + 1) * _NE]
        eqs.append((hg == jnp.max(hg, axis=1, keepdims=True)).astype(bf16))
    eq = jnp.concatenate(eqs, axis=1)
    ii = jax.lax.broadcasted_iota(jnp.int32, (ne, ne), 0)
    jj = jax.lax.broadcasted_iota(jnp.int32, (ne, ne), 1)
    lmat = ((ii // _NE == jj // _NE) & (ii < jj)).astype(bf16)
    s = jnp.dot(eq, lmat, preferred_element_type=f32)
    return jnp.where((eq > 0) & (s == 0.0), h, 0.0)


def _moe_block(x_ref, w1_ref, w2_ref, o_ref):
    w1 = w1_ref[...]
    w2b = w2_ref[...].astype(jnp.bfloat16)
    sub = _BT // _SUB
    hs = [
        jnp.dot(x_ref[pl.ds(p * sub, sub), :], w1,
                preferred_element_type=jnp.float32)
        for p in range(_SUB)
    ]
    zs = [_mask_top1(h) for h in hs]
    for p in range(_SUB):
        o_ref[pl.ds(p * sub, sub), :] = jnp.dot(
            zs[p].astype(jnp.bfloat16), w2b, preferred_element_type=jnp.float32)


@jax.jit
def kernel(x, w1, w2):
    s = x.shape
    xf = x.reshape(-1, _IN)
    t = xf.shape[0]
    w1f = w1.reshape(_IN, _NP * _NE)
    w2f = w2.reshape(_NP * _NE, _OUT)
    out = pl.pallas_call(
        _moe_block,
        grid=(t // _BT,),
        in_specs=[
            pl.BlockSpec((_BT, _IN), lambda i: (i, 0)),
            pl.BlockSpec((_IN, _NP * _NE), lambda i: (0, 0)),
            pl.BlockSpec((_NP * _NE, _OUT), lambda i: (0, 0)),
        ],
        out_specs=pl.BlockSpec((_BT, _OUT), lambda i: (i, 0)),
        out_shape=jax.ShapeDtypeStruct((t, _OUT), jnp.float32),
        compiler_params=pltpu.CompilerParams(
            dimension_semantics=("parallel",),
        ),
    )(xf, w1f, w2f)
    return out.reshape(s[:-1] + (_OUT,))


# speedup vs baseline: 1.1152x; 1.1152x over previous
import functools

import jax
import jax.numpy as jnp
from jax.experimental import pallas as pl
from jax.experimental.pallas import tpu as pltpu

_IN = 768
_OUT = 768
_NP = 4
_NE = 8
_BT = 2048
_SUB = 8


def _mask_top1(h):
    f32 = jnp.float32
    bf16 = jnp.bfloat16
    ne = _NP * _NE
    eqs = []
    for g in range(_NP):
        hg = h[:, g * _NE:(g + 1) * _NE]
        eqs.append((hg == jnp.max(hg, axis=1, keepdims=True)).astype(bf16))
    eq = jnp.concatenate(eqs, axis=1)
    ii = jax.lax.broadcasted_iota(jnp.int32, (ne, ne), 0)
    jj = jax.lax.broadcasted_iota(jnp.int32, (ne, ne), 1)
    lmat = ((ii // _NE == jj // _NE) & (ii < jj)).astype(bf16)
    s = jnp.dot(eq, lmat, preferred_element_type=f32)
    return jnp.where((eq > 0) & (s == 0.0), h, 0.0)


def _moe_block(x_ref, w1_ref, w2_ref, o_ref):
    w1 = w1_ref[...]
    w2b = w2_ref[...].astype(jnp.bfloat16)
    sub = _BT // _SUB
    hs = [
        jnp.dot(x_ref[pl.ds(p * sub, sub), :], w1,
                preferred_element_type=jnp.float32)
        for p in range(_SUB)
    ]
    zs = [_mask_top1(h) for h in hs]
    for p in range(_SUB):
        o_ref[pl.ds(p * sub, sub), :] = jnp.dot(
            zs[p].astype(jnp.bfloat16), w2b, preferred_element_type=jnp.float32)


@jax.jit
def kernel(x, w1, w2):
    s = x.shape
    xf = x.reshape(-1, _IN)
    t = xf.shape[0]
    w1f = w1.reshape(_IN, _NP * _NE)
    w2f = w2.reshape(_NP * _NE, _OUT)
    out = pl.pallas_call(
        _moe_block,
        grid=(t // _BT,),
        in_specs=[
            pl.BlockSpec((_BT, _IN), lambda i: (i, 0)),
            pl.BlockSpec((_IN, _NP * _NE), lambda i: (0, 0)),
            pl.BlockSpec((_NP * _NE, _OUT), lambda i: (0, 0)),
        ],
        out_specs=pl.BlockSpec((_BT, _OUT), lambda i: (i, 0)),
        out_shape=jax.ShapeDtypeStruct((t, _OUT), jnp.float32),
        compiler_params=pltpu.CompilerParams(
            dimension_semantics=("arbitrary",),
        ),
    )(xf, w1f, w2f)
    return out.reshape(s[:-1] + (_OUT,))


# speedup vs baseline: 1.1211x; 1.0053x over previous
import functools

import jax
import jax.numpy as jnp
from jax.experimental import pallas as pl
from jax.experimental.pallas import tpu as pltpu

_IN = 768
_OUT = 768
_NP = 4
_NE = 8
_BT = 2048
_SUB = 16


def _mask_top1(h):
    f32 = jnp.float32
    bf16 = jnp.bfloat16
    ne = _NP * _NE
    eqs = []
    for g in range(_NP):
        hg = h[:, g * _NE:(g + 1) * _NE]
        eqs.append((hg == jnp.max(hg, axis=1, keepdims=True)).astype(bf16))
    eq = jnp.concatenate(eqs, axis=1)
    ii = jax.lax.broadcasted_iota(jnp.int32, (ne, ne), 0)
    jj = jax.lax.broadcasted_iota(jnp.int32, (ne, ne), 1)
    lmat = ((ii // _NE == jj // _NE) & (ii < jj)).astype(bf16)
    s = jnp.dot(eq, lmat, preferred_element_type=f32)
    return jnp.where((eq > 0) & (s == 0.0), h, 0.0)


def _moe_block(x_ref, w1_ref, w2_ref, o_ref):
    w1 = w1_ref[...]
    w2b = w2_ref[...].astype(jnp.bfloat16)
    sub = _BT // _SUB
    hs = [
        jnp.dot(x_ref[pl.ds(p * sub, sub), :], w1,
                preferred_element_type=jnp.float32)
        for p in range(_SUB)
    ]
    zs = [_mask_top1(h) for h in hs]
    for p in range(_SUB):
        o_ref[pl.ds(p * sub, sub), :] = jnp.dot(
            zs[p].astype(jnp.bfloat16), w2b, preferred_element_type=jnp.float32)


@jax.jit
def kernel(x, w1, w2):
    s = x.shape
    xf = x.reshape(-1, _IN)
    t = xf.shape[0]
    w1f = w1.reshape(_IN, _NP * _NE)
    w2f = w2.reshape(_NP * _NE, _OUT)
    out = pl.pallas_call(
        _moe_block,
        grid=(t // _BT,),
        in_specs=[
            pl.BlockSpec((_BT, _IN), lambda i: (i, 0)),
            pl.BlockSpec((_IN, _NP * _NE), lambda i: (0, 0)),
            pl.BlockSpec((_NP * _NE, _OUT), lambda i: (0, 0)),
        ],
        out_specs=pl.BlockSpec((_BT, _OUT), lambda i: (i, 0)),
        out_shape=jax.ShapeDtypeStruct((t, _OUT), jnp.float32),
        compiler_params=pltpu.CompilerParams(
            dimension_semantics=("parallel",),
        ),
    )(xf, w1f, w2f)
    return out.reshape(s[:-1] + (_OUT,))


# speedup vs baseline: 1.1259x; 1.0043x over previous
import jax
import jax.numpy as jnp
from jax.experimental import pallas as pl
from jax.experimental.pallas import tpu as pltpu

_IN = 768
_OUT = 768
_NP = 4
_NE = 8
_BT = 2048
_SUB = 16


def _mask_top1(h):
    f32 = jnp.float32
    bf16 = jnp.bfloat16
    ne = _NP * _NE
    eqs = []
    for g in range(_NP):
        hg = h[:, g * _NE:(g + 1) * _NE]
        eqs.append((hg == jnp.max(hg, axis=1, keepdims=True)).astype(bf16))
    eq = jnp.concatenate(eqs, axis=1)
    ii = jax.lax.broadcasted_iota(jnp.int32, (ne, ne), 0)
    jj = jax.lax.broadcasted_iota(jnp.int32, (ne, ne), 1)
    lmat = ((ii // _NE == jj // _NE) & (ii < jj)).astype(bf16)
    s = jnp.dot(eq, lmat, preferred_element_type=f32)
    return jnp.where((eq > 0) & (s == 0.0), h, 0.0)


def _moe_block(x_ref, w1_ref, w2_ref, o_ref):
    w1 = w1_ref[...]
    w2b = w2_ref[...].astype(jnp.bfloat16)
    sub = _BT // _SUB
    hs = [
        jnp.dot(x_ref[pl.ds(p * sub, sub), :], w1,
                preferred_element_type=jnp.float32)
        for p in range(_SUB)
    ]
    zs = [_mask_top1(h) for h in hs]
    for p in range(_SUB):
        o_ref[pl.ds(p * sub, sub), :] = jnp.dot(
            zs[p].astype(jnp.bfloat16), w2b, preferred_element_type=jnp.float32)


@jax.jit
def kernel(x, w1, w2):
    s = x.shape
    xf = x.reshape(-1, _IN)
    t = xf.shape[0]
    w1f = w1.reshape(_IN, _NP * _NE)
    w2f = w2.reshape(_NP * _NE, _OUT)
    out = pl.pallas_call(
        _moe_block,
        grid=(t // _BT,),
        in_specs=[
            pl.BlockSpec((_BT, _IN), lambda i: (i, 0)),
            pl.BlockSpec((_IN, _NP * _NE), lambda i: (0, 0)),
            pl.BlockSpec((_NP * _NE, _OUT), lambda i: (0, 0)),
        ],
        out_specs=pl.BlockSpec((_BT, _OUT), lambda i: (i, 0)),
        out_shape=jax.ShapeDtypeStruct((t, _OUT), jnp.float32),
        compiler_params=pltpu.CompilerParams(
            dimension_semantics=("parallel",),
        ),
    )(xf, w1f, w2f)
    return out.reshape(s[:-1] + (_OUT,))
